# baseline (speedup 1.0000x reference)
import functools

import jax
import jax.numpy as jnp
from jax import lax
from jax.experimental import pallas as pl
from jax.experimental.pallas import tpu as pltpu
from jax.experimental.pallas import tpu_sc as plsc

NC = 2
NS = 16
NW = NC * NS

N = 10000
E = 320000
D = 128
C = 40

NPAD = 10240
CHUNK = 80
CPB = 25
NBLK = 5
DEPTH = 4
EDGES_PER_TILE = CHUNK * CPB * NBLK
ROWS_PER_TILE = NPAD // NS
RB = 1024


def _make_agg(dz):
    mesh = plsc.VectorSubcoreMesh(core_axis_name="c", subcore_axis_name="s")

    @functools.partial(
        pl.kernel, mesh=mesh,
        compiler_params=pltpu.CompilerParams(use_tc_tiling_on_sc=False),
        out_type=jax.ShapeDtypeStruct((2 * NPAD, dz), jnp.float32),
        scratch_types=[
            pltpu.VMEM((2, CPB, CHUNK), jnp.int32),
            pltpu.VMEM((2, CPB, CHUNK), jnp.int32),
            [pltpu.VMEM((CHUNK, dz), jnp.float32) for _ in range(DEPTH)],
            pltpu.VMEM_SHARED((NPAD, dz), jnp.float32),
            [pltpu.SemaphoreType.DMA for _ in range(DEPTH)],
            [pltpu.SemaphoreType.DMA for _ in range(DEPTH)],
            pltpu.SemaphoreType.DMA,
            pltpu.SemaphoreType.DMA,
        ],
    )
    def agg(z_hbm, edges_hbm, out_hbm,
            src_i, dst_i, rows, acc_sh, gsem, ssem, isem_a, isem_b):
        c = lax.axis_index("c")
        s = lax.axis_index("s")
        wid = s * NC + c
        r0 = s * ROWS_PER_TILE

        def fetch_idx(b, slot, sem):
            pltpu.async_copy(edges_hbm.at[0, wid, b], src_i.at[slot], sem)
            pltpu.async_copy(edges_hbm.at[1, wid, b], dst_i.at[slot], sem)

        def wait_idx(b, slot, sem):
            pltpu.make_async_copy(edges_hbm.at[0, wid, b], src_i.at[slot],
                                  sem).wait()
            pltpu.make_async_copy(edges_hbm.at[1, wid, b], dst_i.at[slot],
                                  sem).wait()

        fetch_idx(0, 0, isem_a)
        pltpu.sync_copy(z_hbm.at[pl.ds(r0, ROWS_PER_TILE)],
                        acc_sh.at[pl.ds(r0, ROWS_PER_TILE)])
        plsc.subcore_barrier()

        NG = 2

        def start_gather(slot, t):
            r = t % DEPTH
            pltpu.async_copy(z_hbm.at[src_i.at[slot, t]], rows[r], gsem[r])

        def wait_gather(slot, t):
            r = t % DEPTH
            pltpu.make_async_copy(z_hbm.at[src_i.at[slot, t]],
                                  rows[r], gsem[r]).wait()

        def start_scatter(slot, t):
            r = t % DEPTH
            pltpu.async_copy(rows[r], acc_sh.at[dst_i.at[slot, t]], ssem[r],
                             add=True)

        def wait_scatter(slot, t):
            r = t % DEPTH
            pltpu.make_async_copy(rows[r], acc_sh.at[dst_i.at[slot, t]],
                                  ssem[r]).wait()

        def run_block(b, slot, other_sem):
            @pl.when(b + 1 < NBLK)
            def _():
                fetch_idx(b + 1, 1 - slot, other_sem)
            for k in range(NG):
                start_gather(slot, k)
            pending = {}
            for t in range(CPB):
                wait_gather(slot, t)
                start_scatter(slot, t)
                pending[t % DEPTH] = (slot, t)
                ng = t + NG
                if ng < CPB:
                    if ng % DEPTH in pending and pending[ng % DEPTH][1] != t:
                        wait_scatter(*pending.pop(ng % DEPTH))
                    start_gather(slot, ng)
            for b_, c_ in sorted(pending.items()):
                wait_scatter(*c_)

        def outer(u, carry):
            b0 = 2 * u
            wait_idx(b0, 0, isem_a)

            run_block(b0, 0, isem_b)
            wait_idx(b0 + 1, 1, isem_b)
            run_block(b0 + 1, 1, isem_a)
            return carry

        lax.fori_loop(0, NBLK // 2, outer, 0)
        wait_idx(NBLK - 1, 0, isem_a)
        run_block(NBLK - 1, 0, isem_b)
        plsc.subcore_barrier()
        pltpu.sync_copy(acc_sh.at[pl.ds(r0, ROWS_PER_TILE)],
                        out_hbm.at[pl.ds(c * NPAD + r0, ROWS_PER_TILE)])

    return agg


_agg128 = _make_agg(128)
_agg48 = _make_agg(48)


def _mm_body(x_ref, w_ref, o_ref):
    o_ref[...] = jnp.dot(x_ref[...], w_ref[...],
                         preferred_element_type=jnp.float32)


def _mm(x, w):
    dz = w.shape[1]
    return pl.pallas_call(
        _mm_body,
        grid=(NPAD // RB,),
        in_specs=[pl.BlockSpec((RB, D), lambda i: (i, 0)),
                  pl.BlockSpec((D, dz), lambda i: (0, 0))],
        out_specs=pl.BlockSpec((RB, dz), lambda i: (i, 0)),
        out_shape=jax.ShapeDtypeStruct((NPAD, dz), jnp.float32),
    )(x, w)


def _mid_body(p_ref, z_ref, b_ref, w_ref, o_ref):
    x = p_ref[0] + p_ref[1] - z_ref[...] + b_ref[...]
    x = jnp.maximum(x, 0.0)
    o_ref[...] = jnp.dot(x, w_ref[...], preferred_element_type=jnp.float32)


def _mid(p, z, b, w):
    din = z.shape[1]
    dz = w.shape[1]
    p3 = p.reshape(2, NPAD, din)
    return pl.pallas_call(
        _mid_body,
        grid=(NPAD // RB,),
        in_specs=[pl.BlockSpec((2, RB, din), lambda i: (0, i, 0)),
                  pl.BlockSpec((RB, din), lambda i: (i, 0)),
                  pl.BlockSpec((1, din), lambda i: (0, 0)),
                  pl.BlockSpec((din, dz), lambda i: (0, 0))],
        out_specs=pl.BlockSpec((RB, dz), lambda i: (i, 0)),
        out_shape=jax.ShapeDtypeStruct((NPAD, dz), jnp.float32),
    )(p3, z, b, w)


def _final_body(p_ref, z_ref, b_ref, o_ref):
    x = p_ref[0] + p_ref[1] - z_ref[...] + b_ref[...]
    col = lax.broadcasted_iota(jnp.int32, x.shape, 1)
    x = jnp.where(col < C, x, -jnp.inf)
    m = jnp.max(x, axis=1, keepdims=True)
    e = jnp.exp(x - m)
    lse = jnp.log(jnp.sum(e, axis=1, keepdims=True))
    o_ref[...] = (x - m - lse)[:, :C]


def _final(p, z, b):
    dz = z.shape[1]
    p3 = p.reshape(2, NPAD, dz)
    return pl.pallas_call(
        _final_body,
        grid=(NPAD // RB,),
        in_specs=[pl.BlockSpec((2, RB, dz), lambda i: (0, i, 0)),
                  pl.BlockSpec((RB, dz), lambda i: (i, 0)),
                  pl.BlockSpec((1, dz), lambda i: (0, 0))],
        out_specs=pl.BlockSpec((RB, C), lambda i: (i, 0)),
        out_shape=jax.ShapeDtypeStruct((NPAD, C), jnp.float32),
    )(p3, z, b)


def kernel(features, edge_index, labels, mask, W0, b0, W1, b1, W2, b2):
    n, d = features.shape
    edges = edge_index.astype(jnp.int32).reshape(2, NW, NBLK, CPB, CHUNK)
    fpad = jnp.pad(features, ((0, NPAD - n), (0, 0)))
    W2p = jnp.pad(W2, ((0, 0), (0, 48 - C)))
    b2p = jnp.pad(b2, (0, 48 - C)).reshape(1, 48)

    z0 = _mm(fpad, W0)
    p0 = _agg128(z0, edges)
    z1 = _mid(p0, z0, b0.reshape(1, -1), W1)
    p1 = _agg128(z1, edges)
    z2 = _mid(p1, z1, b1.reshape(1, -1), W2p)
    p2 = _agg48(z2, edges)
    out = _final(p2, z2, b2p)
    return (out[:n], jnp.asarray(3 * n))


# speedup vs baseline: 1.0993x; 1.0993x over previous
import functools

import jax
import jax.numpy as jnp
from jax import lax
from jax.experimental import pallas as pl
from jax.experimental.pallas import tpu as pltpu
from jax.experimental.pallas import tpu_sc as plsc

NC = 2
NS = 16
NW = NC * NS

N = 10000
E = 320000
D = 128
C = 40

NPAD = 10240
CHUNK = 80
CPB = 25
NBLK = 5
DEPTH = 4
EDGES_PER_TILE = CHUNK * CPB * NBLK
ROWS_PER_TILE = NPAD // NS
RB = 1024


def _make_agg(dz):
    mesh = plsc.VectorSubcoreMesh(core_axis_name="c", subcore_axis_name="s")

    @functools.partial(
        pl.kernel, mesh=mesh,
        compiler_params=pltpu.CompilerParams(use_tc_tiling_on_sc=False),
        out_type=jax.ShapeDtypeStruct((2 * NPAD, dz), jnp.float32),
        scratch_types=[
            pltpu.VMEM((2, CPB, CHUNK), jnp.int32),
            pltpu.VMEM((2, CPB, CHUNK), jnp.int32),
            [pltpu.VMEM((CHUNK, dz), jnp.float32) for _ in range(DEPTH)],
            pltpu.VMEM_SHARED((NPAD, dz), jnp.float32),
            [pltpu.SemaphoreType.DMA for _ in range(DEPTH)],
            [pltpu.SemaphoreType.DMA for _ in range(DEPTH)],
            pltpu.SemaphoreType.DMA,
            pltpu.SemaphoreType.DMA,
        ],
    )
    def agg(z_hbm, edges_hbm, out_hbm,
            src_i, dst_i, rows, acc_sh, gsem, ssem, isem_a, isem_b):
        c = lax.axis_index("c")
        s = lax.axis_index("s")
        wid = s * NC + c
        r0 = s * ROWS_PER_TILE

        def fetch_idx(b, slot, sem):
            pltpu.async_copy(edges_hbm.at[0, wid, b], src_i.at[slot], sem)
            pltpu.async_copy(edges_hbm.at[1, wid, b], dst_i.at[slot], sem)

        def wait_idx(b, slot, sem):
            pltpu.make_async_copy(edges_hbm.at[0, wid, b], src_i.at[slot],
                                  sem).wait()
            pltpu.make_async_copy(edges_hbm.at[1, wid, b], dst_i.at[slot],
                                  sem).wait()

        fetch_idx(0, 0, isem_a)
        pltpu.sync_copy(z_hbm.at[pl.ds(r0, ROWS_PER_TILE)],
                        acc_sh.at[pl.ds(r0, ROWS_PER_TILE)])
        plsc.subcore_barrier()

        NG = 3

        def start_gather(slot, t):
            r = t % DEPTH
            pltpu.async_copy(z_hbm.at[src_i.at[slot, t]], rows[r], gsem[r])

        def wait_gather(slot, t):
            r = t % DEPTH
            pltpu.make_async_copy(z_hbm.at[src_i.at[slot, t]],
                                  rows[r], gsem[r]).wait()

        def start_scatter(slot, t):
            r = t % DEPTH
            pltpu.async_copy(rows[r], acc_sh.at[dst_i.at[slot, t]], ssem[r],
                             add=True)

        def wait_scatter(slot, t):
            r = t % DEPTH
            pltpu.make_async_copy(rows[r], acc_sh.at[dst_i.at[slot, t]],
                                  ssem[r]).wait()

        def run_block(b, slot, other_sem):
            @pl.when(b + 1 < NBLK)
            def _():
                fetch_idx(b + 1, 1 - slot, other_sem)
            for k in range(NG):
                start_gather(slot, k)
            pending = {}
            for t in range(CPB):
                wait_gather(slot, t)
                start_scatter(slot, t)
                pending[t % DEPTH] = (slot, t)
                ng = t + NG
                if ng < CPB:
                    if ng % DEPTH in pending and pending[ng % DEPTH][1] != t:
                        wait_scatter(*pending.pop(ng % DEPTH))
                    start_gather(slot, ng)
            for b_, c_ in sorted(pending.items()):
                wait_scatter(*c_)

        def outer(u, carry):
            b0 = 2 * u
            wait_idx(b0, 0, isem_a)

            run_block(b0, 0, isem_b)
            wait_idx(b0 + 1, 1, isem_b)
            run_block(b0 + 1, 1, isem_a)
            return carry

        lax.fori_loop(0, NBLK // 2, outer, 0)
        wait_idx(NBLK - 1, 0, isem_a)
        run_block(NBLK - 1, 0, isem_b)
        plsc.subcore_barrier()
        pltpu.sync_copy(acc_sh.at[pl.ds(r0, ROWS_PER_TILE)],
                        out_hbm.at[pl.ds(c * NPAD + r0, ROWS_PER_TILE)])

    return agg


_agg128 = _make_agg(128)
_agg48 = _make_agg(48)


def _mm_body(x_ref, w_ref, o_ref):
    o_ref[...] = jnp.dot(x_ref[...], w_ref[...],
                         preferred_element_type=jnp.float32)


def _mm(x, w):
    dz = w.shape[1]
    return pl.pallas_call(
        _mm_body,
        grid=(NPAD // RB,),
        in_specs=[pl.BlockSpec((RB, D), lambda i: (i, 0)),
                  pl.BlockSpec((D, dz), lambda i: (0, 0))],
        out_specs=pl.BlockSpec((RB, dz), lambda i: (i, 0)),
        out_shape=jax.ShapeDtypeStruct((NPAD, dz), jnp.float32),
    )(x, w)


def _mid_body(p_ref, z_ref, b_ref, w_ref, o_ref):
    x = p_ref[0] + p_ref[1] - z_ref[...] + b_ref[...]
    x = jnp.maximum(x, 0.0)
    o_ref[...] = jnp.dot(x, w_ref[...], preferred_element_type=jnp.float32)


def _mid(p, z, b, w):
    din = z.shape[1]
    dz = w.shape[1]
    p3 = p.reshape(2, NPAD, din)
    return pl.pallas_call(
        _mid_body,
        grid=(NPAD // RB,),
        in_specs=[pl.BlockSpec((2, RB, din), lambda i: (0, i, 0)),
                  pl.BlockSpec((RB, din), lambda i: (i, 0)),
                  pl.BlockSpec((1, din), lambda i: (0, 0)),
                  pl.BlockSpec((din, dz), lambda i: (0, 0))],
        out_specs=pl.BlockSpec((RB, dz), lambda i: (i, 0)),
        out_shape=jax.ShapeDtypeStruct((NPAD, dz), jnp.float32),
    )(p3, z, b, w)


def _final_body(p_ref, z_ref, b_ref, o_ref):
    x = p_ref[0] + p_ref[1] - z_ref[...] + b_ref[...]
    col = lax.broadcasted_iota(jnp.int32, x.shape, 1)
    x = jnp.where(col < C, x, -jnp.inf)
    m = jnp.max(x, axis=1, keepdims=True)
    e = jnp.exp(x - m)
    lse = jnp.log(jnp.sum(e, axis=1, keepdims=True))
    o_ref[...] = (x - m - lse)[:, :C]


def _final(p, z, b):
    dz = z.shape[1]
    p3 = p.reshape(2, NPAD, dz)
    return pl.pallas_call(
        _final_body,
        grid=(NPAD // RB,),
        in_specs=[pl.BlockSpec((2, RB, dz), lambda i: (0, i, 0)),
                  pl.BlockSpec((RB, dz), lambda i: (i, 0)),
                  pl.BlockSpec((1, dz), lambda i: (0, 0))],
        out_specs=pl.BlockSpec((RB, C), lambda i: (i, 0)),
        out_shape=jax.ShapeDtypeStruct((NPAD, C), jnp.float32),
    )(p3, z, b)


def kernel(features, edge_index, labels, mask, W0, b0, W1, b1, W2, b2):
    n, d = features.shape
    edges = edge_index.astype(jnp.int32).reshape(2, NW, NBLK, CPB, CHUNK)
    fpad = jnp.pad(features, ((0, NPAD - n), (0, 0)))
    W2p = jnp.pad(W2, ((0, 0), (0, 48 - C)))
    b2p = jnp.pad(b2, (0, 48 - C)).reshape(1, 48)

    z0 = _mm(fpad, W0)
    p0 = _agg128(z0, edges)
    z1 = _mid(p0, z0, b0.reshape(1, -1), W1)
    p1 = _agg128(z1, edges)
    z2 = _mid(p1, z1, b1.reshape(1, -1), W2p)
    p2 = _agg48(z2, edges)
    out = _final(p2, z2, b2p)
    return (out[:n], jnp.asarray(3 * n))


# speedup vs baseline: 1.1198x; 1.0186x over previous
import functools

import jax
import jax.numpy as jnp
from jax import lax
from jax.experimental import pallas as pl
from jax.experimental.pallas import tpu as pltpu
from jax.experimental.pallas import tpu_sc as plsc

NC = 2
NS = 16
NW = NC * NS

N = 10000
E = 320000
D = 128
C = 40

CHUNK = 80
CPB = 25
NBLK = 5
DEPTH = 4
EDGES_PER_TILE = CHUNK * CPB * NBLK
ROWS_PER_TILE = N // NS
RB = 1000


def _make_agg(dz):
    mesh = plsc.VectorSubcoreMesh(core_axis_name="c", subcore_axis_name="s")

    @functools.partial(
        pl.kernel, mesh=mesh,
        compiler_params=pltpu.CompilerParams(use_tc_tiling_on_sc=False),
        out_type=jax.ShapeDtypeStruct((2 * N, dz), jnp.float32),
        scratch_types=[
            pltpu.VMEM((2, CPB, CHUNK), jnp.int32),
            pltpu.VMEM((2, CPB, CHUNK), jnp.int32),
            [pltpu.VMEM((CHUNK, dz), jnp.float32) for _ in range(DEPTH)],
            pltpu.VMEM_SHARED((N, dz), jnp.float32),
            [pltpu.SemaphoreType.DMA for _ in range(DEPTH)],
            pltpu.SemaphoreType.DMA,
            pltpu.SemaphoreType.DMA,
        ],
    )
    def agg(z_hbm, edges_hbm, out_hbm,
            src_i, dst_i, rows, acc_sh, gsem, isem_a, isem_b):
        c = lax.axis_index("c")
        s = lax.axis_index("s")
        wid = s * NC + c
        r0 = s * ROWS_PER_TILE

        def fetch_idx(b, slot, sem):
            pltpu.async_copy(edges_hbm.at[0, wid, b], src_i.at[slot], sem)
            pltpu.async_copy(edges_hbm.at[1, wid, b], dst_i.at[slot], sem)

        def wait_idx(b, slot, sem):
            pltpu.make_async_copy(edges_hbm.at[0, wid, b], src_i.at[slot],
                                  sem).wait()
            pltpu.make_async_copy(edges_hbm.at[1, wid, b], dst_i.at[slot],
                                  sem).wait()

        fetch_idx(0, 0, isem_a)
        pltpu.sync_copy(z_hbm.at[pl.ds(r0, ROWS_PER_TILE)],
                        acc_sh.at[pl.ds(r0, ROWS_PER_TILE)])
        plsc.subcore_barrier()

        def run_block(b, slot, other_sem):
            @pl.when(b + 1 < NBLK)
            def _():
                fetch_idx(b + 1, 1 - slot, other_sem)
            for k in range(DEPTH - 1):
                pltpu.async_copy(z_hbm.at[src_i.at[slot, k]], rows[k], gsem[k])
            for t in range(CPB):
                r = t % DEPTH
                pltpu.make_async_copy(z_hbm.at[src_i.at[slot, t]],
                                      rows[r], gsem[r]).wait()
                pltpu.sync_copy(rows[r], acc_sh.at[dst_i.at[slot, t]], add=True)
                nt = t + DEPTH - 1
                if nt < CPB:
                    pltpu.async_copy(z_hbm.at[src_i.at[slot, nt]],
                                     rows[nt % DEPTH], gsem[nt % DEPTH])

        def outer(u, carry):
            b0 = 2 * u
            wait_idx(b0, 0, isem_a)

            run_block(b0, 0, isem_b)
            wait_idx(b0 + 1, 1, isem_b)
            run_block(b0 + 1, 1, isem_a)
            return carry

        lax.fori_loop(0, NBLK // 2, outer, 0)
        wait_idx(NBLK - 1, 0, isem_a)
        run_block(NBLK - 1, 0, isem_b)
        plsc.subcore_barrier()
        pltpu.sync_copy(acc_sh.at[pl.ds(r0, ROWS_PER_TILE)],
                        out_hbm.at[pl.ds(c * N + r0, ROWS_PER_TILE)])

    return agg


_agg128 = _make_agg(128)
_agg48 = _make_agg(48)


def _mm_body(x_ref, w_ref, o_ref):
    o_ref[...] = jnp.dot(x_ref[...], w_ref[...],
                         preferred_element_type=jnp.float32)


def _mm(x, w):
    dz = w.shape[1]
    return pl.pallas_call(
        _mm_body,
        grid=(N // RB,),
        in_specs=[pl.BlockSpec((RB, D), lambda i: (i, 0)),
                  pl.BlockSpec((D, dz), lambda i: (0, 0))],
        out_specs=pl.BlockSpec((RB, dz), lambda i: (i, 0)),
        out_shape=jax.ShapeDtypeStruct((N, dz), jnp.float32),
    )(x, w)


def _mid_body(p_ref, z_ref, b_ref, w_ref, o_ref):
    x = p_ref[0] + p_ref[1] - z_ref[...] + b_ref[...]
    x = jnp.maximum(x, 0.0)
    o_ref[...] = jnp.dot(x, w_ref[...], preferred_element_type=jnp.float32)


def _mid(p, z, b, w):
    din = z.shape[1]
    dz = w.shape[1]
    p3 = p.reshape(2, N, din)
    return pl.pallas_call(
        _mid_body,
        grid=(N // RB,),
        in_specs=[pl.BlockSpec((2, RB, din), lambda i: (0, i, 0)),
                  pl.BlockSpec((RB, din), lambda i: (i, 0)),
                  pl.BlockSpec((1, din), lambda i: (0, 0)),
                  pl.BlockSpec((din, dz), lambda i: (0, 0))],
        out_specs=pl.BlockSpec((RB, dz), lambda i: (i, 0)),
        out_shape=jax.ShapeDtypeStruct((N, dz), jnp.float32),
    )(p3, z, b, w)


def _final_body(p_ref, z_ref, b_ref, o_ref):
    x = p_ref[0] + p_ref[1] - z_ref[...] + b_ref[...]
    col = lax.broadcasted_iota(jnp.int32, x.shape, 1)
    x = jnp.where(col < C, x, -jnp.inf)
    m = jnp.max(x, axis=1, keepdims=True)
    e = jnp.exp(x - m)
    lse = jnp.log(jnp.sum(e, axis=1, keepdims=True))
    o_ref[...] = (x - m - lse)[:, :C]


def _final(p, z, b):
    dz = z.shape[1]
    p3 = p.reshape(2, N, dz)
    return pl.pallas_call(
        _final_body,
        grid=(N // RB,),
        in_specs=[pl.BlockSpec((2, RB, dz), lambda i: (0, i, 0)),
                  pl.BlockSpec((RB, dz), lambda i: (i, 0)),
                  pl.BlockSpec((1, dz), lambda i: (0, 0))],
        out_specs=pl.BlockSpec((RB, C), lambda i: (i, 0)),
        out_shape=jax.ShapeDtypeStruct((N, C), jnp.float32),
    )(p3, z, b)


def kernel(features, edge_index, labels, mask, W0, b0, W1, b1, W2, b2):
    n, d = features.shape
    edges = edge_index.astype(jnp.int32).reshape(2, NW, NBLK, CPB, CHUNK)
    W2p = jnp.pad(W2, ((0, 0), (0, 48 - C)))
    b2p = jnp.pad(b2, (0, 48 - C)).reshape(1, 48)

    z0 = _mm(features, W0)
    p0 = _agg128(z0, edges)
    z1 = _mid(p0, z0, b0.reshape(1, -1), W1)
    p1 = _agg128(z1, edges)
    z2 = _mid(p1, z1, b1.reshape(1, -1), W2p)
    p2 = _agg48(z2, edges)
    out = _final(p2, z2, b2p)
    return (out, jnp.asarray(3 * n))


# speedup vs baseline: 1.1523x; 1.0290x over previous
import functools

import jax
import jax.numpy as jnp
from jax import lax
from jax.experimental import pallas as pl
from jax.experimental.pallas import tpu as pltpu
from jax.experimental.pallas import tpu_sc as plsc

NC = 2
NS = 16
NW = NC * NS

N = 10000
E = 320000
D = 128
C = 40

CHUNK = 80
CPB = 25
NBLK = 5
DEPTH = 4
EDGES_PER_TILE = CHUNK * CPB * NBLK
ROWS_PER_TILE = N // NS
RB = 2000


def _make_agg(dz):
    mesh = plsc.VectorSubcoreMesh(core_axis_name="c", subcore_axis_name="s")

    @functools.partial(
        pl.kernel, mesh=mesh,
        compiler_params=pltpu.CompilerParams(use_tc_tiling_on_sc=False),
        out_type=jax.ShapeDtypeStruct((2 * N, dz), jnp.float32),
        scratch_types=[
            pltpu.VMEM((2, CPB, CHUNK), jnp.int32),
            pltpu.VMEM((2, CPB, CHUNK), jnp.int32),
            [pltpu.VMEM((CHUNK, dz), jnp.float32) for _ in range(DEPTH)],
            pltpu.VMEM_SHARED((N, dz), jnp.float32),
            [pltpu.SemaphoreType.DMA for _ in range(DEPTH)],
            pltpu.SemaphoreType.DMA,
            pltpu.SemaphoreType.DMA,
        ],
    )
    def agg(z_hbm, edges_hbm, out_hbm,
            src_i, dst_i, rows, acc_sh, gsem, isem_a, isem_b):
        c = lax.axis_index("c")
        s = lax.axis_index("s")
        wid = s * NC + c
        r0 = s * ROWS_PER_TILE

        def fetch_idx(b, slot, sem):
            pltpu.async_copy(edges_hbm.at[0, wid, b], src_i.at[slot], sem)
            pltpu.async_copy(edges_hbm.at[1, wid, b], dst_i.at[slot], sem)

        def wait_idx(b, slot, sem):
            pltpu.make_async_copy(edges_hbm.at[0, wid, b], src_i.at[slot],
                                  sem).wait()
            pltpu.make_async_copy(edges_hbm.at[1, wid, b], dst_i.at[slot],
                                  sem).wait()

        fetch_idx(0, 0, isem_a)
        pltpu.sync_copy(z_hbm.at[pl.ds(r0, ROWS_PER_TILE)],
                        acc_sh.at[pl.ds(r0, ROWS_PER_TILE)])
        plsc.subcore_barrier()

        def run_block(b, slot, other_sem):
            @pl.when(b + 1 < NBLK)
            def _():
                fetch_idx(b + 1, 1 - slot, other_sem)
            for k in range(DEPTH - 1):
                pltpu.async_copy(z_hbm.at[src_i.at[slot, k]], rows[k], gsem[k])
            for t in range(CPB):
                r = t % DEPTH
                pltpu.make_async_copy(z_hbm.at[src_i.at[slot, t]],
                                      rows[r], gsem[r]).wait()
                pltpu.sync_copy(rows[r], acc_sh.at[dst_i.at[slot, t]], add=True)
                nt = t + DEPTH - 1
                if nt < CPB:
                    pltpu.async_copy(z_hbm.at[src_i.at[slot, nt]],
                                     rows[nt % DEPTH], gsem[nt % DEPTH])

        def outer(u, carry):
            b0 = 2 * u
            wait_idx(b0, 0, isem_a)

            run_block(b0, 0, isem_b)
            wait_idx(b0 + 1, 1, isem_b)
            run_block(b0 + 1, 1, isem_a)
            return carry

        lax.fori_loop(0, NBLK // 2, outer, 0)
        wait_idx(NBLK - 1, 0, isem_a)
        run_block(NBLK - 1, 0, isem_b)
        plsc.subcore_barrier()
        pltpu.sync_copy(acc_sh.at[pl.ds(r0, ROWS_PER_TILE)],
                        out_hbm.at[pl.ds(c * N + r0, ROWS_PER_TILE)])

    return agg


_agg128 = _make_agg(128)
_agg48 = _make_agg(48)


def _mm_body(x_ref, w_ref, o_ref):
    o_ref[...] = jnp.dot(x_ref[...], w_ref[...],
                         preferred_element_type=jnp.float32)


def _mm(x, w):
    dz = w.shape[1]
    return pl.pallas_call(
        _mm_body,
        grid=(N // RB,),
        in_specs=[pl.BlockSpec((RB, D), lambda i: (i, 0)),
                  pl.BlockSpec((D, dz), lambda i: (0, 0))],
        out_specs=pl.BlockSpec((RB, dz), lambda i: (i, 0)),
        out_shape=jax.ShapeDtypeStruct((N, dz), jnp.float32),
    )(x, w)


def _mid_body(p_ref, z_ref, b_ref, w_ref, o_ref):
    x = p_ref[0] + p_ref[1] - z_ref[...] + b_ref[...]
    x = jnp.maximum(x, 0.0)
    o_ref[...] = jnp.dot(x, w_ref[...], preferred_element_type=jnp.float32)


def _mid(p, z, b, w):
    din = z.shape[1]
    dz = w.shape[1]
    p3 = p.reshape(2, N, din)
    return pl.pallas_call(
        _mid_body,
        grid=(N // RB,),
        in_specs=[pl.BlockSpec((2, RB, din), lambda i: (0, i, 0)),
                  pl.BlockSpec((RB, din), lambda i: (i, 0)),
                  pl.BlockSpec((1, din), lambda i: (0, 0)),
                  pl.BlockSpec((din, dz), lambda i: (0, 0))],
        out_specs=pl.BlockSpec((RB, dz), lambda i: (i, 0)),
        out_shape=jax.ShapeDtypeStruct((N, dz), jnp.float32),
    )(p3, z, b, w)


def _final_body(p_ref, z_ref, b_ref, o_ref):
    x = p_ref[0] + p_ref[1] - z_ref[...] + b_ref[...]
    col = lax.broadcasted_iota(jnp.int32, x.shape, 1)
    x = jnp.where(col < C, x, -jnp.inf)
    m = jnp.max(x, axis=1, keepdims=True)
    e = jnp.exp(x - m)
    lse = jnp.log(jnp.sum(e, axis=1, keepdims=True))
    o_ref[...] = (x - m - lse)[:, :C]


def _final(p, z, b):
    dz = z.shape[1]
    p3 = p.reshape(2, N, dz)
    return pl.pallas_call(
        _final_body,
        grid=(N // RB,),
        in_specs=[pl.BlockSpec((2, RB, dz), lambda i: (0, i, 0)),
                  pl.BlockSpec((RB, dz), lambda i: (i, 0)),
                  pl.BlockSpec((1, dz), lambda i: (0, 0))],
        out_specs=pl.BlockSpec((RB, C), lambda i: (i, 0)),
        out_shape=jax.ShapeDtypeStruct((N, C), jnp.float32),
    )(p3, z, b)


def kernel(features, edge_index, labels, mask, W0, b0, W1, b1, W2, b2):
    n, d = features.shape
    edges = edge_index.astype(jnp.int32).reshape(2, NW, NBLK, CPB, CHUNK)
    W2p = jnp.pad(W2, ((0, 0), (0, 48 - C)))
    b2p = jnp.pad(b2, (0, 48 - C)).reshape(1, 48)

    z0 = _mm(features, W0)
    p0 = _agg128(z0, edges)
    z1 = _mid(p0, z0, b0.reshape(1, -1), W1)
    p1 = _agg128(z1, edges)
    z2 = _mid(p1, z1, b1.reshape(1, -1), W2p)
    p2 = _agg48(z2, edges)
    out = _final(p2, z2, b2p)
    return (out, jnp.asarray(3 * n))


# speedup vs baseline: 1.1713x; 1.0166x over previous
import functools

import jax
import jax.numpy as jnp
from jax import lax
from jax.experimental import pallas as pl
from jax.experimental.pallas import tpu as pltpu
from jax.experimental.pallas import tpu_sc as plsc

NC = 2
NS = 16
NW = NC * NS

N = 10000
E = 320000
D = 128
C = 40

CHUNK = 80
CPB = 25
NBLK = 5
DEPTH = 4
EDGES_PER_TILE = CHUNK * CPB * NBLK
ROWS_PER_TILE = N // NS
RB = 5000


def _make_agg(dz):
    mesh = plsc.VectorSubcoreMesh(core_axis_name="c", subcore_axis_name="s")

    @functools.partial(
        pl.kernel, mesh=mesh,
        compiler_params=pltpu.CompilerParams(use_tc_tiling_on_sc=False),
        out_type=jax.ShapeDtypeStruct((2 * N, dz), jnp.float32),
        scratch_types=[
            pltpu.VMEM((2, CPB, CHUNK), jnp.int32),
            pltpu.VMEM((2, CPB, CHUNK), jnp.int32),
            [pltpu.VMEM((CHUNK, dz), jnp.float32) for _ in range(DEPTH)],
            pltpu.VMEM_SHARED((N, dz), jnp.float32),
            [pltpu.SemaphoreType.DMA for _ in range(DEPTH)],
            pltpu.SemaphoreType.DMA,
            pltpu.SemaphoreType.DMA,
        ],
    )
    def agg(z_hbm, edges_hbm, out_hbm,
            src_i, dst_i, rows, acc_sh, gsem, isem_a, isem_b):
        c = lax.axis_index("c")
        s = lax.axis_index("s")
        wid = s * NC + c
        r0 = s * ROWS_PER_TILE

        def fetch_idx(b, slot, sem):
            pltpu.async_copy(edges_hbm.at[0, wid, b], src_i.at[slot], sem)
            pltpu.async_copy(edges_hbm.at[1, wid, b], dst_i.at[slot], sem)

        def wait_idx(b, slot, sem):
            pltpu.make_async_copy(edges_hbm.at[0, wid, b], src_i.at[slot],
                                  sem).wait()
            pltpu.make_async_copy(edges_hbm.at[1, wid, b], dst_i.at[slot],
                                  sem).wait()

        fetch_idx(0, 0, isem_a)
        pltpu.sync_copy(z_hbm.at[pl.ds(r0, ROWS_PER_TILE)],
                        acc_sh.at[pl.ds(r0, ROWS_PER_TILE)])
        plsc.subcore_barrier()

        def run_block(b, slot, other_sem):
            @pl.when(b + 1 < NBLK)
            def _():
                fetch_idx(b + 1, 1 - slot, other_sem)
            for k in range(DEPTH - 1):
                pltpu.async_copy(z_hbm.at[src_i.at[slot, k]], rows[k], gsem[k])
            for t in range(CPB):
                r = t % DEPTH
                pltpu.make_async_copy(z_hbm.at[src_i.at[slot, t]],
                                      rows[r], gsem[r]).wait()
                pltpu.sync_copy(rows[r], acc_sh.at[dst_i.at[slot, t]], add=True)
                nt = t + DEPTH - 1
                if nt < CPB:
                    pltpu.async_copy(z_hbm.at[src_i.at[slot, nt]],
                                     rows[nt % DEPTH], gsem[nt % DEPTH])

        def outer(u, carry):
            b0 = 2 * u
            wait_idx(b0, 0, isem_a)

            run_block(b0, 0, isem_b)
            wait_idx(b0 + 1, 1, isem_b)
            run_block(b0 + 1, 1, isem_a)
            return carry

        lax.fori_loop(0, NBLK // 2, outer, 0)
        wait_idx(NBLK - 1, 0, isem_a)
        run_block(NBLK - 1, 0, isem_b)
        plsc.subcore_barrier()
        pltpu.sync_copy(acc_sh.at[pl.ds(r0, ROWS_PER_TILE)],
                        out_hbm.at[pl.ds(c * N + r0, ROWS_PER_TILE)])

    return agg


_agg128 = _make_agg(128)
_agg48 = _make_agg(48)


def _mm_body(x_ref, w_ref, o_ref):
    o_ref[...] = jnp.dot(x_ref[...], w_ref[...],
                         preferred_element_type=jnp.float32)


def _mm(x, w):
    dz = w.shape[1]
    return pl.pallas_call(
        _mm_body,
        grid=(N // RB,),
        in_specs=[pl.BlockSpec((RB, D), lambda i: (i, 0)),
                  pl.BlockSpec((D, dz), lambda i: (0, 0))],
        out_specs=pl.BlockSpec((RB, dz), lambda i: (i, 0)),
        out_shape=jax.ShapeDtypeStruct((N, dz), jnp.float32),
    )(x, w)


def _mid_body(p_ref, z_ref, b_ref, w_ref, o_ref):
    x = p_ref[0] + p_ref[1] - z_ref[...] + b_ref[...]
    x = jnp.maximum(x, 0.0)
    o_ref[...] = jnp.dot(x, w_ref[...], preferred_element_type=jnp.float32)


def _mid(p, z, b, w):
    din = z.shape[1]
    dz = w.shape[1]
    p3 = p.reshape(2, N, din)
    return pl.pallas_call(
        _mid_body,
        grid=(N // RB,),
        in_specs=[pl.BlockSpec((2, RB, din), lambda i: (0, i, 0)),
                  pl.BlockSpec((RB, din), lambda i: (i, 0)),
                  pl.BlockSpec((1, din), lambda i: (0, 0)),
                  pl.BlockSpec((din, dz), lambda i: (0, 0))],
        out_specs=pl.BlockSpec((RB, dz), lambda i: (i, 0)),
        out_shape=jax.ShapeDtypeStruct((N, dz), jnp.float32),
    )(p3, z, b, w)


def _final_body(p_ref, z_ref, b_ref, o_ref):
    x = p_ref[0] + p_ref[1] - z_ref[...] + b_ref[...]
    col = lax.broadcasted_iota(jnp.int32, x.shape, 1)
    x = jnp.where(col < C, x, -jnp.inf)
    m = jnp.max(x, axis=1, keepdims=True)
    e = jnp.exp(x - m)
    lse = jnp.log(jnp.sum(e, axis=1, keepdims=True))
    o_ref[...] = (x - m - lse)[:, :C]


def _final(p, z, b):
    dz = z.shape[1]
    p3 = p.reshape(2, N, dz)
    return pl.pallas_call(
        _final_body,
        grid=(N // RB,),
        in_specs=[pl.BlockSpec((2, RB, dz), lambda i: (0, i, 0)),
                  pl.BlockSpec((RB, dz), lambda i: (i, 0)),
                  pl.BlockSpec((1, dz), lambda i: (0, 0))],
        out_specs=pl.BlockSpec((RB, C), lambda i: (i, 0)),
        out_shape=jax.ShapeDtypeStruct((N, C), jnp.float32),
    )(p3, z, b)


def kernel(features, edge_index, labels, mask, W0, b0, W1, b1, W2, b2):
    n, d = features.shape
    edges = edge_index.astype(jnp.int32).reshape(2, NW, NBLK, CPB, CHUNK)
    W2p = jnp.pad(W2, ((0, 0), (0, 48 - C)))
    b2p = jnp.pad(b2, (0, 48 - C)).reshape(1, 48)

    z0 = _mm(features, W0)
    p0 = _agg128(z0, edges)
    z1 = _mid(p0, z0, b0.reshape(1, -1), W1)
    p1 = _agg128(z1, edges)
    z2 = _mid(p1, z1, b1.reshape(1, -1), W2p)
    p2 = _agg48(z2, edges)
    out = _final(p2, z2, b2p)
    return (out, jnp.asarray(3 * n))
